# baseline (speedup 1.0000x reference)
import math

import jax
import jax.numpy as jnp
from jax import lax
from jax.experimental import pallas as pl
from jax.experimental.pallas import tpu as pltpu
from jax.experimental.pallas import tpu_sc as plsc

VOCAB = 1000000
D = 64
L = 16
NC, NS = 2, 16
NW = NC * NS
BATCH = 4096
HIST = 200
B = BATCH * HIST
PER_W = B // NW
CHUNK = 512
NCHUNK = PER_W // CHUNK

MAX_NORM = (1.0 - 0.001) / math.sqrt(1.0)
INV_MAX_NORM = 1.0 / MAX_NORM


def _rsqrt_nr(s):
    i = lax.bitcast_convert_type(s, jnp.int32)
    i = jnp.int32(0x5F3759DF) - lax.shift_right_arithmetic(i, 1)
    y = lax.bitcast_convert_type(i, jnp.float32)
    y = y * (1.5 - 0.5 * (s * y) * y)
    y = y * (1.5 - 0.5 * (s * y) * y)
    return y


def _recip_nr(d):
    i = lax.bitcast_convert_type(d, jnp.int32)
    z = lax.bitcast_convert_type(jnp.int32(0x7EF127EA) - i, jnp.float32)
    z = z * (2.0 - d * z)
    z = z * (2.0 - d * z)
    return z


def _body(
    ids_hbm, weight_hbm, out_hbm,
    idx0, idx1, rows0, rows1, fac_v, rsem, osem,
):
    wid = lax.axis_index("s") * NC + lax.axis_index("c")
    w_base = wid * PER_W
    idxs = (idx0, idx1)
    rows = (rows0, rows1)

    def compute(rows_v):
        def ss_body(r, _):
            v = [rows_v[r, pl.ds(k * L, L)] for k in range(D // L)]
            ss = v[0] * v[0]
            for k in range(1, D // L):
                ss = ss + v[k] * v[k]
            s = jnp.sum(ss)
            rs = _rsqrt_nr(s)
            norm = s * rs
            scale = jnp.minimum(norm * INV_MAX_NORM, 1.0)
            f = _recip_nr(scale + 1e-8)
            for k in range(D // L):
                rows_v[r, pl.ds(k * L, L)] = v[k] * f
            return 0

        lax.fori_loop(0, CHUNK, ss_body, 0, unroll=4)

    pltpu.sync_copy(ids_hbm.at[pl.ds(w_base, CHUNK)], idx0)
    pltpu.async_copy(weight_hbm.at[idx0], rows0, rsem)

    def pair_body(cp, _):
        for b in range(2):
            c = 2 * cp + b
            base = w_base + c * CHUNK
            rb, ib = rows[b], idxs[b]
            nrb, nib = rows[1 - b], idxs[1 - b]

            @pl.when(c >= 1)
            def _():
                pltpu.make_async_copy(
                    nrb, out_hbm.at[pl.ds(base - CHUNK, CHUNK)], osem
                ).wait()

            @pl.when(c < NCHUNK - 1)
            def _():
                pltpu.sync_copy(
                    ids_hbm.at[pl.ds(base + CHUNK, CHUNK)], nib
                )
                pltpu.async_copy(weight_hbm.at[nib], nrb, rsem)

            pltpu.make_async_copy(weight_hbm.at[ib], rb, rsem).wait()
            compute(rb)
            pltpu.async_copy(rb, out_hbm.at[pl.ds(base, CHUNK)], osem)
        return 0

    lax.fori_loop(0, NCHUNK // 2, pair_body, 0)
    pltpu.make_async_copy(
        rows1, out_hbm.at[pl.ds(w_base + (NCHUNK - 1) * CHUNK, CHUNK)], osem
    ).wait()


@jax.jit
def _run(ids_flat, weight):
    mesh = plsc.VectorSubcoreMesh(core_axis_name="c", subcore_axis_name="s")
    return pl.kernel(
        _body,
        out_type=jax.ShapeDtypeStruct((B, D), jnp.float32),
        mesh=mesh,
        compiler_params=pltpu.CompilerParams(
            needs_layout_passes=False, use_tc_tiling_on_sc=False
        ),
        scratch_types=[
            pltpu.VMEM((CHUNK,), jnp.int32),
            pltpu.VMEM((CHUNK,), jnp.int32),
            pltpu.VMEM((CHUNK, D), jnp.float32),
            pltpu.VMEM((CHUNK, D), jnp.float32),
            pltpu.SMEM((CHUNK,), jnp.float32),
            pltpu.SemaphoreType.DMA,
            pltpu.SemaphoreType.DMA,
        ],
    )(ids_flat, weight)


def kernel(input_ids, weight):
    ids_flat = input_ids.reshape(B)
    out = _run(ids_flat, weight)
    return out.reshape(BATCH, HIST, D)


# speedup vs baseline: 1.0545x; 1.0545x over previous
import math

import jax
import jax.numpy as jnp
from jax import lax
from jax.experimental import pallas as pl
from jax.experimental.pallas import tpu as pltpu
from jax.experimental.pallas import tpu_sc as plsc

VOCAB = 1000000
D = 64
L = 16
NC, NS = 2, 16
NW = NC * NS
BATCH = 4096
HIST = 200
B = BATCH * HIST
PER_W = B // NW
CHUNK = 512
NCHUNK = PER_W // CHUNK

MAX_NORM = (1.0 - 0.001) / math.sqrt(1.0)
INV_MAX_NORM = 1.0 / MAX_NORM


def _rsqrt_nr(s):
    i = lax.bitcast_convert_type(s, jnp.int32)
    i = jnp.int32(0x5F3759DF) - lax.shift_right_arithmetic(i, 1)
    y = lax.bitcast_convert_type(i, jnp.float32)
    y = y * (1.5 - 0.5 * (s * y) * y)
    y = y * (1.5 - 0.5 * (s * y) * y)
    return y


def _recip_nr(d):
    i = lax.bitcast_convert_type(d, jnp.int32)
    z = lax.bitcast_convert_type(jnp.int32(0x7EF127EA) - i, jnp.float32)
    z = z * (2.0 - d * z)
    z = z * (2.0 - d * z)
    return z


def _body(
    ids_hbm, weight_hbm, out_hbm,
    idx0, idx1, rows0, rows1, fac_v, rsem, osem,
):
    wid = lax.axis_index("s") * NC + lax.axis_index("c")
    w_base = wid * PER_W
    idxs = (idx0, idx1)
    rows = (rows0, rows1)

    def compute(rows_v):
        def ss_body(r, _):
            ss = jnp.zeros((L,), jnp.float32)
            for k in range(D // L):
                v = rows_v[r, pl.ds(k * L, L)]
                ss = ss + v * v
            s = jnp.sum(ss)
            rs = _rsqrt_nr(s)
            norm = s * rs
            scale = jnp.minimum(norm * INV_MAX_NORM, 1.0)
            fac_v[r] = _recip_nr(scale + 1e-8)
            return 0

        lax.fori_loop(0, CHUNK, ss_body, 0, unroll=4)

        def row_body(r, _):
            f = fac_v[r]
            for k in range(D // L):
                rows_v[r, pl.ds(k * L, L)] = rows_v[r, pl.ds(k * L, L)] * f
            return 0

        lax.fori_loop(0, CHUNK, row_body, 0, unroll=4)

    pltpu.sync_copy(ids_hbm.at[pl.ds(w_base, CHUNK)], idx0)
    pltpu.async_copy(weight_hbm.at[idx0], rows0, rsem)

    def pair_body(cp, _):
        for b in range(2):
            c = 2 * cp + b
            base = w_base + c * CHUNK
            rb, ib = rows[b], idxs[b]
            nrb, nib = rows[1 - b], idxs[1 - b]

            @pl.when(c >= 1)
            def _():
                pltpu.make_async_copy(
                    nrb, out_hbm.at[pl.ds(base - CHUNK, CHUNK)], osem
                ).wait()

            @pl.when(c < NCHUNK - 1)
            def _():
                pltpu.sync_copy(
                    ids_hbm.at[pl.ds(base + CHUNK, CHUNK)], nib
                )
                pltpu.async_copy(weight_hbm.at[nib], nrb, rsem)

            pltpu.make_async_copy(weight_hbm.at[ib], rb, rsem).wait()
            compute(rb)
            pltpu.async_copy(rb, out_hbm.at[pl.ds(base, CHUNK)], osem)
        return 0

    lax.fori_loop(0, NCHUNK // 2, pair_body, 0)
    pltpu.make_async_copy(
        rows1, out_hbm.at[pl.ds(w_base + (NCHUNK - 1) * CHUNK, CHUNK)], osem
    ).wait()


@jax.jit
def _run(ids_flat, weight):
    mesh = plsc.VectorSubcoreMesh(core_axis_name="c", subcore_axis_name="s")
    return pl.kernel(
        _body,
        out_type=jax.ShapeDtypeStruct((B, D), jnp.float32),
        mesh=mesh,
        compiler_params=pltpu.CompilerParams(
            needs_layout_passes=False, use_tc_tiling_on_sc=False
        ),
        scratch_types=[
            pltpu.VMEM((CHUNK,), jnp.int32),
            pltpu.VMEM((CHUNK,), jnp.int32),
            pltpu.VMEM((CHUNK, D), jnp.float32),
            pltpu.VMEM((CHUNK, D), jnp.float32),
            pltpu.SMEM((CHUNK,), jnp.float32),
            pltpu.SemaphoreType.DMA,
            pltpu.SemaphoreType.DMA,
        ],
    )(ids_flat, weight)


def kernel(input_ids, weight):
    ids_flat = input_ids.reshape(B)
    out = _run(ids_flat, weight)
    return out.reshape(BATCH, HIST, D)
